# baseline (speedup 1.0000x reference)
import functools

import jax
import jax.numpy as jnp
from jax.experimental import pallas as pl
from jax.experimental.pallas import tpu as pltpu

N = 10000
E = 320000
F = 128
FE = 16
H = 8
DH = F // H
FM = 256

ROWS = 400


def _ln(x, scale=None, bias=None, eps=1e-6):
    m = jnp.mean(x, axis=-1, keepdims=True)
    v = jnp.var(x, axis=-1, keepdims=True)
    y = (x - m) / jnp.sqrt(v + eps)
    if scale is not None:
        y = y * scale
    if bias is not None:
        y = y + bias
    return y


def _mlp_body(x_ref, g2_ref, b2m_ref, a2_ref, W1_ref, b1_ref, W2_ref, b2_ref,
              o_ref):
    x = x_ref[...]
    xp = _ln(x) * (1.0 + g2_ref[...]) + b2m_ref[...]
    h = jax.nn.gelu(xp @ W1_ref[...] + b1_ref[...][None, :])
    h = h @ W2_ref[...] + b2_ref[...][None, :]
    o_ref[...] = x + h * a2_ref[...]


def _mlp(x, gamma2, beta2, alpha2, W1, b1, W2, b2):
    grid = (N // ROWS,)
    blk = pl.BlockSpec((ROWS, F), lambda i: (i, 0))
    blkm = pl.BlockSpec((ROWS, FM), lambda i: (i, 0))
    full = lambda shape: pl.BlockSpec(shape, lambda i: tuple(0 for _ in shape))
    return pl.pallas_call(
        _mlp_body,
        grid=grid,
        in_specs=[blk, blk, blk, blk, full((F, FM)), full((FM,)),
                  full((FM, F)), full((F,))],
        out_specs=blk,
        out_shape=jax.ShapeDtypeStruct((N, F), jnp.float32),
    )(x, gamma2, beta2, alpha2, W1, b1, W2, b2)


def kernel(features_nodes, features_edges, features_time, cutoff_value,
           senders, receivers, ln_c_scale, ln_c_bias, W_ada, b_ada, Wq, Wk,
           Wv, Wo, Wrk, Wrv, W1, b1, W2, b2):
    x = jnp.squeeze(features_nodes, axis=(1, 2))
    e = jnp.squeeze(features_edges, axis=(1, 2))
    c = jnp.squeeze(features_time, axis=(1, 2))
    c = _ln(c, ln_c_scale, ln_c_bias)
    c = jax.nn.silu(c)
    mod = c @ W_ada + b_ada
    gamma1, beta1, alpha1, gamma2, beta2, alpha2 = jnp.split(mod, 6, axis=-1)
    x_pre = _ln(x) * (1.0 + gamma1) + beta1
    q = (x_pre @ Wq).reshape(N, H, DH)
    k = (x_pre @ Wk).reshape(N, H, DH)
    v = (x_pre @ Wv).reshape(N, H, DH)
    rk = (e @ Wrk).reshape(E, H, DH)
    rv = (e @ Wrv).reshape(E, H, DH)
    qd = q[receivers]
    ks_ = k[senders]
    vs = v[senders]
    logits = jnp.sum(qd * (ks_ + rk), axis=-1) / jnp.sqrt(DH)
    m = jax.ops.segment_max(logits, receivers, num_segments=N)
    m = jnp.where(jnp.isfinite(m), m, 0.0)
    w = jnp.exp(logits - m[receivers]) * cutoff_value[:, None]
    den = jax.ops.segment_sum(w, receivers, num_segments=N)
    attn = w / (den[receivers] + 1e-9)
    out = jax.ops.segment_sum(attn[:, :, None] * (vs + rv), receivers,
                              num_segments=N)
    out = out.reshape(N, F) @ Wo
    cut_sum = jax.ops.segment_sum(cutoff_value, receivers, num_segments=N)
    cnt = jax.ops.segment_sum(jnp.ones_like(cutoff_value), receivers,
                              num_segments=N)
    cut_mean = cut_sum / jnp.maximum(cnt, 1.0)
    post_att = jnp.where(cut_mean.reshape(-1, 1) < 1e-5, x_pre, out)
    x = x + post_att * alpha1
    x = _mlp(x, gamma2, beta2, alpha2, W1, b1, W2, b2)
    return x[:, None, None, :]
